# baseline (speedup 1.0000x reference)
import functools

import jax
import jax.numpy as jnp
from jax.experimental import pallas as pl


def _pe_add_block(dur_ref, pe_ref, x_ref, o_ref, *, sblk, batches, words):
    sidx = pl.program_id(0)
    rows = sblk * batches

    dur = dur_ref[...].astype(jnp.float32)
    tri = (
        jax.lax.broadcasted_iota(jnp.int32, (words, words), 0)
        <= jax.lax.broadcasted_iota(jnp.int32, (words, words), 1)
    ).astype(jnp.float32)
    csum_in = jnp.dot(dur, tri, preferred_element_type=jnp.float32)
    csum_ex = csum_in - dur

    ci_t = jnp.broadcast_to(csum_in[None], (sblk, batches, words)).reshape(
        rows, words
    )
    ce_t = jnp.broadcast_to(csum_ex[None], (sblk, batches, words)).reshape(
        rows, words
    )

    pos = (
        (jax.lax.broadcasted_iota(jnp.int32, (rows, words), 0) // batches)
        + sidx * sblk
    ).astype(jnp.float32)

    onehot = ((pos >= ce_t) & (pos < ci_t)).astype(jnp.bfloat16)
    pe_bf = pe_ref[...].astype(jnp.bfloat16)
    add = jnp.dot(onehot, pe_bf, preferred_element_type=jnp.float32)

    chans = pe_ref.shape[1]
    xb = x_ref[...].reshape(rows, chans)
    o_ref[...] = (xb + add).reshape(sblk, batches, chans)


def kernel(x, pe, text_duration, train):
    del train
    S, B, C = x.shape
    _, W = text_duration.shape
    pe_trunc = pe[:W, :]
    sblk = 64
    grid = (S // sblk,)

    return pl.pallas_call(
        functools.partial(_pe_add_block, sblk=sblk, batches=B, words=W),
        grid=grid,
        in_specs=[
            pl.BlockSpec((B, W), lambda s: (0, 0)),
            pl.BlockSpec((W, C), lambda s: (0, 0)),
            pl.BlockSpec((sblk, B, C), lambda s: (s, 0, 0)),
        ],
        out_specs=pl.BlockSpec((sblk, B, C), lambda s: (s, 0, 0)),
        out_shape=jax.ShapeDtypeStruct((S, B, C), x.dtype),
    )(text_duration.astype(jnp.int32), pe_trunc, x)
